# Initial kernel scaffold; baseline (speedup 1.0000x reference)
import jax
import jax.numpy as jnp
from jax.experimental import pallas as pl


def kernel(query_pts, query_times, buffer_voxel_index, static_features, dynamic_features, time_embeddings, f1_Wqkv, f1_bqkv, f1_Wo, f1_bo, f2_Wqkv, f2_bqkv, f2_Wo, f2_bo):
    raise NotImplementedError("write your pallas kernel here")


# speedup vs baseline: 14.2544x; 14.2544x over previous
import functools
import math

import jax
import jax.numpy as jnp
from jax import lax
from jax.experimental import pallas as pl
from jax.experimental.pallas import tpu as pltpu
from jax.experimental.pallas import tpu_sc as plsc

RES = 0.1
TABLE = 1 << 20
D = 120
T = 201
H = 8
DH = D // H
P0, P1, P2 = 73856093, 19349669, 83492791

NC, NS = 2, 16
NW = NC * NS
C = 128
BH = 4096
BA = 1024


def _hash_body(qp_ref, h_ref):
    t = qp_ref[...] / jnp.float32(RES)
    g = jnp.floor(t).astype(jnp.int32)
    s = g[0] * P0 + g[1] * P1 + g[2] * P2
    h_ref[...] = jnp.bitwise_and(s, TABLE - 1)


def _hash_call(qpt, mp):
    grid = mp // BH
    return pl.pallas_call(
        _hash_body,
        grid=(grid,),
        in_specs=[pl.BlockSpec((3, BH), lambda i: (0, i))],
        out_specs=pl.BlockSpec((BH,), lambda i: (i,)),
        out_shape=jax.ShapeDtypeStruct((mp,), jnp.int32),
    )(qpt)


def _sc_body(nchunk, h_hbm, buf_hbm, st_hbm, dy_hbm,
             stg_hbm, dyg_hbm, vv_hbm,
             hloc, vv, sv, strow, dyrow, sem0, sem1, sem2):
    cid = lax.axis_index("c")
    sid = lax.axis_index("s")
    wid = sid * NC + cid

    def body(j, carry):
        base = (wid * nchunk + j) * C
        pltpu.sync_copy(h_hbm.at[pl.ds(base, C)], hloc)
        pltpu.async_copy(buf_hbm.at[hloc], vv, sem0).wait()
        for i in range(C // 16):
            v = vv[pl.ds(i * 16, 16)]
            sv[pl.ds(i * 16, 16)] = jnp.maximum(v, 0)
        c1 = pltpu.async_copy(st_hbm.at[sv], strow, sem1)
        c2 = pltpu.async_copy(dy_hbm.at[sv], dyrow, sem2)
        c1.wait()
        c2.wait()
        pltpu.sync_copy(strow, stg_hbm.at[pl.ds(base, C)])
        pltpu.sync_copy(dyrow, dyg_hbm.at[pl.ds(base, C)])
        pltpu.sync_copy(vv, vv_hbm.at[pl.ds(base, C)])
        return carry

    lax.fori_loop(0, nchunk, body, 0)


def _sc_call(h, buf, st, dy, mp):
    nchunk = mp // (NW * C)
    mesh = plsc.VectorSubcoreMesh(core_axis_name="c", subcore_axis_name="s")
    return pl.kernel(
        functools.partial(_sc_body, nchunk),
        out_type=(
            jax.ShapeDtypeStruct((mp, D), jnp.float32),
            jax.ShapeDtypeStruct((mp, D), jnp.float32),
            jax.ShapeDtypeStruct((mp,), jnp.int32),
        ),
        mesh=mesh,
        compiler_params=pltpu.CompilerParams(use_tc_tiling_on_sc=False),
        scratch_types=[
            pltpu.VMEM((C,), jnp.int32),
            pltpu.VMEM((C,), jnp.int32),
            pltpu.VMEM((C,), jnp.int32),
            pltpu.VMEM((C, D), jnp.float32),
            pltpu.VMEM((C, D), jnp.float32),
            pltpu.SemaphoreType.DMA,
            pltpu.SemaphoreType.DMA,
            pltpu.SemaphoreType.DMA,
        ],
    )(h, buf, st, dy)


def _fuse(a, b, wq, wk, wv, bq, bk, bv, wo, bo, bd):
    f32 = jnp.float32
    qa = jnp.dot(a, wq, preferred_element_type=f32) + bq
    ka = jnp.dot(a, wk, preferred_element_type=f32) + bk
    va = jnp.dot(a, wv, preferred_element_type=f32) + bv
    kb = jnp.dot(b, wk, preferred_element_type=f32) + bk
    vb = jnp.dot(b, wv, preferred_element_type=f32) + bv
    p0 = jnp.dot(qa * ka, bd, preferred_element_type=f32)
    p1 = jnp.dot(qa * kb, bd, preferred_element_type=f32)
    mx = jnp.maximum(p0, p1)
    e0 = jnp.exp(p0 - mx)
    e1 = jnp.exp(p1 - mx)
    r = 1.0 / (e0 + e1)
    o = (e0 * r) * va + (e1 * r) * vb
    return jnp.dot(o, wo, preferred_element_type=f32) + bo


def _attn_body(dy_ref, st_ref, t_ref, v_ref, te_ref,
               wq1, wk1, wv1, bq1, bk1, bv1, wo1, bo1,
               wq2, wk2, wv2, bq2, bk2, bv2, wo2, bo2,
               out_ref):
    f32 = jnp.float32
    t = jnp.remainder(t_ref[...], T)
    oh = (t == lax.broadcasted_iota(jnp.int32, (BA, T), 1)).astype(f32)
    te = jnp.dot(oh, te_ref[...], preferred_element_type=f32)
    ri = lax.broadcasted_iota(jnp.int32, (D, D), 0) // DH
    ci = lax.broadcasted_iota(jnp.int32, (D, D), 1) // DH
    bd = jnp.where(ri == ci, f32(1.0 / math.sqrt(DH)), f32(0.0))
    cond = _fuse(dy_ref[...], te,
                 wq1[...], wk1[...], wv1[...],
                 bq1[...], bk1[...], bv1[...], wo1[...], bo1[...], bd)
    fused = _fuse(st_ref[...], cond,
                  wq2[...], wk2[...], wv2[...],
                  bq2[...], bk2[...], bv2[...], wo2[...], bo2[...], bd)
    out_ref[...] = jnp.where(v_ref[...] >= 0, fused, f32(0.0))


def _attn_call(m, dyg, stg, t2d, v2d, te, w1, w2):
    grid = (m + BA - 1) // BA
    full2d = lambda shape: pl.BlockSpec(shape, lambda i: (0, 0))
    wspecs = [full2d((D, D)), full2d((D, D)), full2d((D, D)),
              full2d((1, D)), full2d((1, D)), full2d((1, D)),
              full2d((D, D)), full2d((1, D))]
    return pl.pallas_call(
        _attn_body,
        grid=(grid,),
        in_specs=[
            pl.BlockSpec((BA, D), lambda i: (i, 0)),
            pl.BlockSpec((BA, D), lambda i: (i, 0)),
            pl.BlockSpec((BA, 1), lambda i: (i, 0)),
            pl.BlockSpec((BA, 1), lambda i: (i, 0)),
            full2d((T, D)),
        ] + wspecs + wspecs,
        out_specs=pl.BlockSpec((BA, D), lambda i: (i, 0)),
        out_shape=jax.ShapeDtypeStruct((m, D), jnp.float32),
    )(dyg, stg, t2d, v2d, te, *w1, *w2)


def kernel(query_pts, query_times, buffer_voxel_index, static_features,
           dynamic_features, time_embeddings,
           f1_Wqkv, f1_bqkv, f1_Wo, f1_bo,
           f2_Wqkv, f2_bqkv, f2_Wo, f2_bo):
    m = query_pts.shape[0]
    unit = NW * C
    mp = ((m + unit - 1) // unit) * unit

    qpt = jnp.transpose(query_pts).astype(jnp.float32)
    qpt = jnp.pad(qpt, ((0, 0), (0, mp - m)))
    times = jnp.pad(query_times.astype(jnp.int32), (0, mp - m))
    t2d = times.reshape(mp, 1)
    buf = buffer_voxel_index.astype(jnp.int32)

    h = _hash_call(qpt, mp)
    stg, dyg, vidx = _sc_call(h, buf, static_features.astype(jnp.float32),
                              dynamic_features.astype(jnp.float32), mp)
    v2d = vidx.reshape(mp, 1)

    def wpack(wqkv, bqkv, wo, bo):
        return (wqkv[:, :D], wqkv[:, D:2 * D], wqkv[:, 2 * D:],
                bqkv[:D].reshape(1, D), bqkv[D:2 * D].reshape(1, D),
                bqkv[2 * D:].reshape(1, D), wo, bo.reshape(1, D))

    w1 = wpack(f1_Wqkv.astype(jnp.float32), f1_bqkv.astype(jnp.float32),
               f1_Wo.astype(jnp.float32), f1_bo.astype(jnp.float32))
    w2 = wpack(f2_Wqkv.astype(jnp.float32), f2_bqkv.astype(jnp.float32),
               f2_Wo.astype(jnp.float32), f2_bo.astype(jnp.float32))

    te = time_embeddings.astype(jnp.float32)
    return _attn_call(m, dyg, stg, t2d, v2d, te, w1, w2)


# speedup vs baseline: 15.1114x; 1.0601x over previous
import functools
import math

import jax
import jax.numpy as jnp
from jax import lax
from jax.experimental import pallas as pl
from jax.experimental.pallas import tpu as pltpu
from jax.experimental.pallas import tpu_sc as plsc

RES = 0.1
TABLE = 1 << 20
D = 120
DP = 128
LA = 120
T = 201
H = 8
DH = D // H
P0, P1, P2 = 73856093, 19349669, 83492791

NC, NS = 2, 16
NW = NC * NS
C = 128
GA = 8
BH = 4096
BA = 1024
UNIT = NW * GA * C


def _hash_body(qx_ref, qy_ref, qz_ref, h_ref):
    res = jnp.float32(RES)
    gx = jnp.floor(qx_ref[...] / res).astype(jnp.int32)
    gy = jnp.floor(qy_ref[...] / res).astype(jnp.int32)
    gz = jnp.floor(qz_ref[...] / res).astype(jnp.int32)
    s = gx * P0 + gy * P1 + gz * P2
    h_ref[...] = jnp.bitwise_and(s, TABLE - 1)


def _hash_call(qx2, qy2, qz2, nrow):
    rb = BH // 128
    grid = nrow // rb
    spec = pl.BlockSpec((rb, 128), lambda i: (i, 0))
    return pl.pallas_call(
        _hash_body,
        grid=(grid,),
        in_specs=[spec, spec, spec],
        out_specs=spec,
        out_shape=jax.ShapeDtypeStruct((nrow, 128), jnp.int32),
    )(qx2, qy2, qz2)


def _aux_rows(vvr, tlr, svr, auxr):
    for i in range(GA):
        for j in range(128 // 16):
            sl = pl.ds(j * 16, 16)
            v = vvr[i, sl]
            t = tlr[i, sl]
            svr[i, sl] = jnp.maximum(v, 0)
            auxr[i, sl] = jnp.where(v >= 0, t, -1)


def _lookup_body(ngroup, h_hbm, buf_hbm, t_hbm, sv_hbm, aux_hbm,
                 h0, h1, t0, t1, vv0, vv1, sv0, sv1, ax0, ax1, sem0, sem1):
    cid = lax.axis_index("c")
    sid = lax.axis_index("s")
    wid = sid * NC + cid

    def pair(p, carry):
        r0 = (wid * ngroup + 2 * p) * GA
        r1 = r0 + GA
        pltpu.sync_copy(h_hbm.at[pl.ds(r0, GA)], h0)
        cps0 = [pltpu.async_copy(buf_hbm.at[h0.at[i]], vv0.at[i], sem0)
                for i in range(GA)]
        pltpu.sync_copy(h_hbm.at[pl.ds(r1, GA)], h1)
        cps1 = [pltpu.async_copy(buf_hbm.at[h1.at[i]], vv1.at[i], sem1)
                for i in range(GA)]
        pltpu.sync_copy(t_hbm.at[pl.ds(r0, GA)], t0)
        pltpu.sync_copy(t_hbm.at[pl.ds(r1, GA)], t1)
        for cp in cps0:
            cp.wait()
        _aux_rows(vv0, t0, sv0, ax0)
        pltpu.sync_copy(sv0, sv_hbm.at[pl.ds(r0, GA)])
        pltpu.sync_copy(ax0, aux_hbm.at[pl.ds(r0, GA)])
        for cp in cps1:
            cp.wait()
        _aux_rows(vv1, t1, sv1, ax1)
        pltpu.sync_copy(sv1, sv_hbm.at[pl.ds(r1, GA)])
        pltpu.sync_copy(ax1, aux_hbm.at[pl.ds(r1, GA)])
        return carry

    lax.fori_loop(0, ngroup // 2, pair, 0)


def _lookup_call(h2d, buf, t2d, nrow):
    ngroup = nrow // (NW * GA)
    mesh = plsc.VectorSubcoreMesh(core_axis_name="c", subcore_axis_name="s")
    i2d = lambda: pltpu.VMEM((GA, 128), jnp.int32)
    return pl.kernel(
        functools.partial(_lookup_body, ngroup),
        out_type=(
            jax.ShapeDtypeStruct((nrow, 128), jnp.int32),
            jax.ShapeDtypeStruct((nrow, 128), jnp.int32),
        ),
        mesh=mesh,
        compiler_params=pltpu.CompilerParams(use_tc_tiling_on_sc=False, needs_layout_passes=False),
        scratch_types=[i2d(), i2d(), i2d(), i2d(), i2d(), i2d(),
                       i2d(), i2d(), i2d(), i2d(),
                       pltpu.SemaphoreType.DMA, pltpu.SemaphoreType.DMA],
    )(h2d, buf, t2d)


def _embed_aux(auxl, strow):
    cidx = jnp.full((16,), LA, jnp.int32)
    for j in range(C // 16):
        ridx = lax.broadcasted_iota(jnp.int32, (16,), 0) + j * 16
        af = auxl[pl.ds(j * 16, 16)].astype(jnp.float32)
        plsc.store_scatter(strow, [ridx, cidx], af)


def _rows_body(nchunk, sv_hbm, aux_hbm, st_hbm, dy_hbm, stg_hbm, dyg_hbm,
               svl0, svl1, axl0, axl1, str0, str1, dyr0, dyr1,
               sem_s0, sem_s1, sem_d0, sem_d1):
    cid = lax.axis_index("c")
    sid = lax.axis_index("s")
    wid = sid * NC + cid

    def pair(p, carry):
        c0 = wid * nchunk + 2 * p
        c1 = c0 + 1
        pltpu.sync_copy(sv_hbm.at[c0], svl0)
        cs0 = pltpu.async_copy(st_hbm.at[svl0], str0, sem_s0)
        cd0 = pltpu.async_copy(dy_hbm.at[svl0], dyr0, sem_d0)
        pltpu.sync_copy(sv_hbm.at[c1], svl1)
        cs1 = pltpu.async_copy(st_hbm.at[svl1], str1, sem_s1)
        cd1 = pltpu.async_copy(dy_hbm.at[svl1], dyr1, sem_d1)
        pltpu.sync_copy(aux_hbm.at[c0], axl0)
        pltpu.sync_copy(aux_hbm.at[c1], axl1)
        cs0.wait()
        cd0.wait()
        _embed_aux(axl0, str0)
        pltpu.sync_copy(str0, stg_hbm.at[pl.ds(c0 * C, C)])
        pltpu.sync_copy(dyr0, dyg_hbm.at[pl.ds(c0 * C, C)])
        cs1.wait()
        cd1.wait()
        _embed_aux(axl1, str1)
        pltpu.sync_copy(str1, stg_hbm.at[pl.ds(c1 * C, C)])
        pltpu.sync_copy(dyr1, dyg_hbm.at[pl.ds(c1 * C, C)])
        return carry

    lax.fori_loop(0, nchunk // 2, pair, 0)


def _rows_call(sv2d, aux2d, stp, dyp, mp):
    nchunk = mp // (NW * C)
    mesh = plsc.VectorSubcoreMesh(core_axis_name="c", subcore_axis_name="s")
    rows = lambda: pltpu.VMEM((C, DP), jnp.float32)
    i1d = lambda: pltpu.VMEM((C,), jnp.int32)
    return pl.kernel(
        functools.partial(_rows_body, nchunk),
        out_type=(
            jax.ShapeDtypeStruct((mp, DP), jnp.float32),
            jax.ShapeDtypeStruct((mp, DP), jnp.float32),
        ),
        mesh=mesh,
        compiler_params=pltpu.CompilerParams(needs_layout_passes=False),
        scratch_types=[i1d(), i1d(), i1d(), i1d(),
                       rows(), rows(), rows(), rows(),
                       pltpu.SemaphoreType.DMA, pltpu.SemaphoreType.DMA,
                       pltpu.SemaphoreType.DMA, pltpu.SemaphoreType.DMA],
    )(sv2d, aux2d, stp, dyp)


def _fuse(a, b, wq, wk, wv, bq, bk, bv, wo, bo, bd, scale):
    f32 = jnp.float32
    b16 = b.astype(jnp.bfloat16)
    qa = jnp.dot(a, wq, preferred_element_type=f32) + bq
    ka = jnp.dot(a, wk, preferred_element_type=f32) + bk
    va = jnp.dot(a, wv, preferred_element_type=f32) + bv
    kb = jnp.dot(b16, wk, preferred_element_type=f32) + bk
    vb = jnp.dot(b16, wv, preferred_element_type=f32) + bv
    p0 = jnp.dot((qa * ka).astype(jnp.bfloat16), bd,
                 preferred_element_type=f32) * scale
    p1 = jnp.dot((qa * kb).astype(jnp.bfloat16), bd,
                 preferred_element_type=f32) * scale
    mx = jnp.maximum(p0, p1)
    e0 = jnp.exp(p0 - mx)
    e1 = jnp.exp(p1 - mx)
    r = 1.0 / (e0 + e1)
    o = (e0 * r) * va + (e1 * r) * vb
    return jnp.dot(o.astype(jnp.bfloat16), wo, preferred_element_type=f32) + bo


def _attn_body(dy_ref, st_ref, te_ref,
               wq1, wk1, wv1, bq1, bk1, bv1, wo1, bo1,
               wq2, wk2, wv2, bq2, bk2, bv2, wo2, bo2,
               out_ref):
    f32 = jnp.float32
    stb = st_ref[...]
    aux = stb[:, LA:LA + 1]
    ti = aux.astype(jnp.int32)
    oh = (ti == lax.broadcasted_iota(jnp.int32, (BA, T), 1))
    te = jnp.dot(oh.astype(jnp.bfloat16), te_ref[...],
                 preferred_element_type=f32)
    ri = lax.broadcasted_iota(jnp.int32, (D, D), 0) // DH
    ci = lax.broadcasted_iota(jnp.int32, (D, D), 1) // DH
    bd = (ri == ci).astype(jnp.bfloat16)
    scale = f32(1.0 / math.sqrt(DH))
    a1 = dy_ref[...][:, :D].astype(jnp.bfloat16)
    a2 = stb[:, :D].astype(jnp.bfloat16)
    cond = _fuse(a1, te,
                 wq1[...], wk1[...], wv1[...],
                 bq1[...], bk1[...], bv1[...], wo1[...], bo1[...], bd, scale)
    fused = _fuse(a2, cond,
                  wq2[...], wk2[...], wv2[...],
                  bq2[...], bk2[...], bv2[...], wo2[...], bo2[...], bd, scale)
    out_ref[...] = jnp.where(aux >= 0.0, fused, f32(0.0))


def _attn_call(m, dyg, stg, te16, w1, w2):
    grid = (m + BA - 1) // BA
    full2d = lambda shape: pl.BlockSpec(shape, lambda i: (0, 0))
    wspecs = [full2d((D, D)), full2d((D, D)), full2d((D, D)),
              full2d((1, D)), full2d((1, D)), full2d((1, D)),
              full2d((D, D)), full2d((1, D))]
    return pl.pallas_call(
        _attn_body,
        grid=(grid,),
        in_specs=[
            pl.BlockSpec((BA, DP), lambda i: (i, 0)),
            pl.BlockSpec((BA, DP), lambda i: (i, 0)),
            full2d((T, D)),
        ] + wspecs + wspecs,
        out_specs=pl.BlockSpec((BA, D), lambda i: (i, 0)),
        out_shape=jax.ShapeDtypeStruct((m, D), jnp.float32),
    )(dyg, stg, te16, *w1, *w2)


def kernel(query_pts, query_times, buffer_voxel_index, static_features,
           dynamic_features, time_embeddings,
           f1_Wqkv, f1_bqkv, f1_Wo, f1_bo,
           f2_Wqkv, f2_bqkv, f2_Wo, f2_bo):
    m = query_pts.shape[0]
    mp = ((m + UNIT - 1) // UNIT) * UNIT
    nrow = mp // 128

    pts = query_pts.astype(jnp.float32)
    qx2 = jnp.pad(pts[:, 0], (0, mp - m)).reshape(nrow, 128)
    qy2 = jnp.pad(pts[:, 1], (0, mp - m)).reshape(nrow, 128)
    qz2 = jnp.pad(pts[:, 2], (0, mp - m)).reshape(nrow, 128)
    t2d = jnp.pad(query_times.astype(jnp.int32),
                  (0, mp - m)).reshape(nrow, 128)
    buf = buffer_voxel_index.astype(jnp.int32)
    stp = jnp.pad(static_features.astype(jnp.float32), ((0, 0), (0, DP - D)))
    dyp = jnp.pad(dynamic_features.astype(jnp.float32), ((0, 0), (0, DP - D)))

    h2d = _hash_call(qx2, qy2, qz2, nrow)
    sv2d, aux2d = _lookup_call(h2d, buf, t2d, nrow)
    stg, dyg = _rows_call(sv2d, aux2d, stp, dyp, mp)

    def wpack(wqkv, bqkv, wo, bo):
        w16 = wqkv.astype(jnp.bfloat16)
        b32 = bqkv.astype(jnp.float32)
        return (w16[:, :D], w16[:, D:2 * D], w16[:, 2 * D:],
                b32[:D].reshape(1, D), b32[D:2 * D].reshape(1, D),
                b32[2 * D:].reshape(1, D),
                wo.astype(jnp.bfloat16), bo.astype(jnp.float32).reshape(1, D))

    w1 = wpack(f1_Wqkv, f1_bqkv, f1_Wo, f1_bo)
    w2 = wpack(f2_Wqkv, f2_bqkv, f2_Wo, f2_bo)
    te16 = time_embeddings.astype(jnp.bfloat16)

    return _attn_call(m, dyg, stg, te16, w1, w2)
